# baseline (speedup 1.0000x reference)
import functools

import jax
import jax.numpy as jnp
from jax import lax
from jax.experimental import pallas as pl
from jax.experimental.pallas import tpu as pltpu
from jax.experimental.pallas import tpu_sc as plsc

N = 10000
NP = 10240
NSTEPS_ROW = 160
E = 320000
D = 128
NC, NS = 2, 16
NW = NC * NS
C = 128
NCHUNK = E // C
ITERS = -(-NCHUNK // NW)
DW = 16


def _gather_sc(x_attr3, col_i32):
    mesh = plsc.VectorSubcoreMesh(
        core_axis_name="c", subcore_axis_name="s",
        num_cores=NC, num_subcores=NS)

    @functools.partial(
        pl.kernel,
        out_type=jax.ShapeDtypeStruct((E, 1, D), jnp.float32),
        mesh=mesh,
        scratch_types=[
            pltpu.VMEM((C,), jnp.int32),
            pltpu.VMEM((C, 1, D), jnp.float32),
            pltpu.SemaphoreType.DMA,
        ],
    )
    def g(x_hbm, col_hbm, out_hbm, colv, gbuf, sem):
        cid = lax.axis_index("c")
        sid = lax.axis_index("s")
        wid = sid * NC + cid

        def body(it, carry):
            cidx = it * NW + wid

            @pl.when(cidx < NCHUNK)
            def _():
                e0 = cidx * C
                pltpu.sync_copy(col_hbm.at[pl.ds(e0, C)], colv)
                pltpu.async_copy(x_hbm.at[colv], gbuf, sem).wait()
                pltpu.sync_copy(gbuf, out_hbm.at[pl.ds(e0, C)])

            return carry

        lax.fori_loop(0, ITERS, body, 0)

    return g(x_attr3, col_i32)


def _segsum_tc(g2, row2, x_news_pad, w1a, w1b, b1, w2, b2):
    EB = 2000
    NSTEP = E // EB

    def body(row_ref, g_ref, x_ref, w1a_ref, w1b_ref, b1_ref, w2_ref, b2_ref,
             o_ref, acc, dacc):
        step = pl.program_id(0)

        @pl.when(step == 0)
        def _():
            acc[...] = jnp.zeros_like(acc)
            dacc[...] = jnp.zeros_like(dacc)

        def upd(i, carry):
            r = row_ref[0, 0, i]
            acc[pl.ds(r, 1), :] += g_ref[pl.ds(i, 1), :]
            dacc[pl.ds(r, 1), :] += 1.0
            return carry

        lax.fori_loop(0, EB, upd, 0)

        @pl.when(step == NSTEP - 1)
        def _():
            agr = acc[...] / (dacc[:, 0:1] + 1e-8)
            h = jnp.tanh(
                jnp.dot(x_ref[...], w1a_ref[...],
                        preferred_element_type=jnp.float32)
                + jnp.dot(agr, w1b_ref[...],
                          preferred_element_type=jnp.float32)
                + b1_ref[...])
            o_ref[...] = (
                jnp.dot(h, w2_ref[...], preferred_element_type=jnp.float32)
                + b2_ref[...])

    return pl.pallas_call(
        body,
        grid=(NSTEP,),
        in_specs=[
            pl.BlockSpec((1, 1, EB), lambda i: (i, 0, 0),
                         memory_space=pltpu.SMEM),
            pl.BlockSpec((EB, D), lambda i: (i, 0)),
            pl.BlockSpec((NP, D), lambda i: (0, 0)),
            pl.BlockSpec((D, D), lambda i: (0, 0)),
            pl.BlockSpec((D, D), lambda i: (0, 0)),
            pl.BlockSpec((1, D), lambda i: (0, 0)),
            pl.BlockSpec((D, D), lambda i: (0, 0)),
            pl.BlockSpec((1, D), lambda i: (0, 0)),
        ],
        out_specs=pl.BlockSpec((NP, D), lambda i: (0, 0)),
        out_shape=jax.ShapeDtypeStruct((NP, D), jnp.float32),
        scratch_shapes=[
            pltpu.VMEM((NP, D), jnp.float32),
            pltpu.VMEM((NP, DW), jnp.float32),
        ],
    )(row2, g2, x_news_pad, w1a, w1b, b1, w2, b2)


def kernel(x_news, x_attr, edge_index, W1, b1, W2, b2):
    row = edge_index[0].astype(jnp.int32)
    col = edge_index[1].astype(jnp.int32)
    g = _gather_sc(x_attr[:, None, :], col)
    g2 = g.reshape(E, D)
    x_news_pad = jnp.pad(x_news, ((0, NP - N), (0, 0)))
    out_pad = _segsum_tc(g2, row.reshape(NSTEPS_ROW, 1, -1), x_news_pad,
                         W1[:D], W1[D:], b1[None, :], W2, b2[None, :])
    return out_pad[:N]


# speedup vs baseline: 1.1220x; 1.1220x over previous
import functools

import jax
import jax.numpy as jnp
from jax import lax
from jax.experimental import pallas as pl
from jax.experimental.pallas import tpu as pltpu
from jax.experimental.pallas import tpu_sc as plsc

N = 10000
NP = 10240
NSTEPS_ROW = 160
E = 320000
D = 128
NC, NS = 2, 16
NW = NC * NS
C = 128
NCHUNK = E // C
ITERS = -(-NCHUNK // NW)
DW = 16


def _gather_sc(x_attr3, col_i32):
    mesh = plsc.VectorSubcoreMesh(
        core_axis_name="c", subcore_axis_name="s",
        num_cores=NC, num_subcores=NS)

    @functools.partial(
        pl.kernel,
        out_type=jax.ShapeDtypeStruct((E, 1, D), jnp.float32),
        mesh=mesh,
        scratch_types=[
            pltpu.VMEM((C,), jnp.int32),
            pltpu.VMEM((C, 1, D), jnp.float32),
            pltpu.SemaphoreType.DMA,
        ],
    )
    def g(x_hbm, col_hbm, out_hbm, colv, gbuf, sem):
        cid = lax.axis_index("c")
        sid = lax.axis_index("s")
        wid = sid * NC + cid

        def body(it, carry):
            cidx = it * NW + wid

            @pl.when(cidx < NCHUNK)
            def _():
                e0 = cidx * C
                pltpu.sync_copy(col_hbm.at[pl.ds(e0, C)], colv)
                pltpu.async_copy(x_hbm.at[colv], gbuf, sem).wait()
                pltpu.sync_copy(gbuf, out_hbm.at[pl.ds(e0, C)])

            return carry

        lax.fori_loop(0, ITERS, body, 0)

    return g(x_attr3, col_i32)


def _segsum_tc(g2, row2, x_news_pad, w1a, w1b, b1, w2, b2):
    EB = 2000
    NSTEP = E // EB

    U = 2

    def body(row_ref, g_ref, x_ref, w1a_ref, w1b_ref, b1_ref, w2_ref, b2_ref,
             o_ref, *scratch):
        accs = scratch[:U]
        dacc = scratch[U]
        step = pl.program_id(0)

        @pl.when(step == 0)
        def _():
            for u in range(U):
                accs[u][...] = jnp.zeros_like(accs[u])
            dacc[...] = jnp.zeros_like(dacc)

        def upd(i, carry):
            for u in range(U):
                e = i * U + u
                r = row_ref[0, 0, e]
                accs[u][pl.ds(r, 1), :] += g_ref[pl.ds(e, 1), :]
                dacc[pl.ds(r, 1), :] += 1.0
            return carry

        lax.fori_loop(0, EB // U, upd, 0)

        @pl.when(step == NSTEP - 1)
        def _():
            acc_t = accs[0][...] + accs[1][...]
            deg_t = dacc[:, 0:1]
            agr = acc_t / (deg_t + 1e-8)
            h = jnp.tanh(
                jnp.dot(x_ref[...], w1a_ref[...],
                        preferred_element_type=jnp.float32)
                + jnp.dot(agr, w1b_ref[...],
                          preferred_element_type=jnp.float32)
                + b1_ref[...])
            o_ref[...] = (
                jnp.dot(h, w2_ref[...], preferred_element_type=jnp.float32)
                + b2_ref[...])

    return pl.pallas_call(
        body,
        grid=(NSTEP,),
        in_specs=[
            pl.BlockSpec((1, 1, EB), lambda i: (i, 0, 0),
                         memory_space=pltpu.SMEM),
            pl.BlockSpec((EB, D), lambda i: (i, 0)),
            pl.BlockSpec((NP, D), lambda i: (0, 0)),
            pl.BlockSpec((D, D), lambda i: (0, 0)),
            pl.BlockSpec((D, D), lambda i: (0, 0)),
            pl.BlockSpec((1, D), lambda i: (0, 0)),
            pl.BlockSpec((D, D), lambda i: (0, 0)),
            pl.BlockSpec((1, D), lambda i: (0, 0)),
        ],
        out_specs=pl.BlockSpec((NP, D), lambda i: (0, 0)),
        out_shape=jax.ShapeDtypeStruct((NP, D), jnp.float32),
        scratch_shapes=(
            [pltpu.VMEM((NP, D), jnp.float32)] * 2
            + [pltpu.VMEM((NP, DW), jnp.float32)]
        ),
    )(row2, g2, x_news_pad, w1a, w1b, b1, w2, b2)


def kernel(x_news, x_attr, edge_index, W1, b1, W2, b2):
    row = edge_index[0].astype(jnp.int32)
    col = edge_index[1].astype(jnp.int32)
    g = _gather_sc(x_attr[:, None, :], col)
    g2 = g.reshape(E, D)
    x_news_pad = jnp.pad(x_news, ((0, NP - N), (0, 0)))
    out_pad = _segsum_tc(g2, row.reshape(NSTEPS_ROW, 1, -1), x_news_pad,
                         W1[:D], W1[D:], b1[None, :], W2, b2[None, :])
    return out_pad[:N]


# speedup vs baseline: 1.4913x; 1.3291x over previous
import functools

import jax
import jax.numpy as jnp
from jax import lax
from jax.experimental import pallas as pl
from jax.experimental.pallas import tpu as pltpu
from jax.experimental.pallas import tpu_sc as plsc

N = 10000
NP = 10240
NSTEPS_ROW = 160
E = 320000
D = 128
NC, NS = 2, 16
NW = NC * NS
C = 128
NCHUNK = E // C
ITERS = -(-NCHUNK // NW)
DW = 16


def _gather_sc(x_attr3, col_i32):
    mesh = plsc.VectorSubcoreMesh(
        core_axis_name="c", subcore_axis_name="s",
        num_cores=NC, num_subcores=NS)

    @functools.partial(
        pl.kernel,
        out_type=jax.ShapeDtypeStruct((E, 1, D), jnp.float32),
        mesh=mesh,
        scratch_types=[
            pltpu.VMEM((C,), jnp.int32),
            pltpu.VMEM((C, 1, D), jnp.float32),
            pltpu.SemaphoreType.DMA,
        ],
    )
    def g(x_hbm, col_hbm, out_hbm, colv, gbuf, sem):
        cid = lax.axis_index("c")
        sid = lax.axis_index("s")
        wid = sid * NC + cid

        def body(it, carry):
            cidx = it * NW + wid

            @pl.when(cidx < NCHUNK)
            def _():
                e0 = cidx * C
                pltpu.sync_copy(col_hbm.at[pl.ds(e0, C)], colv)
                pltpu.async_copy(x_hbm.at[colv], gbuf, sem).wait()
                pltpu.sync_copy(gbuf, out_hbm.at[pl.ds(e0, C)])

            return carry

        lax.fori_loop(0, ITERS, body, 0)

    return g(x_attr3, col_i32)


def _segsum_tc(g2, row2, x_news_pad, w1a, w1b, b1, w2, b2):
    EB = 2000
    NSTEP = E // EB

    U = 4

    def body(row_ref, g_ref, x_ref, w1a_ref, w1b_ref, b1_ref, w2_ref, b2_ref,
             o_ref, *scratch):
        accs = scratch[:U]
        dacc = scratch[U]
        step = pl.program_id(0)

        @pl.when(step == 0)
        def _():
            for u in range(U):
                accs[u][...] = jnp.zeros_like(accs[u])
            dacc[...] = jnp.zeros_like(dacc)

        def upd(i, carry):
            for u in range(U):
                e = i * U + u
                r = row_ref[0, 0, e]
                accs[u][pl.ds(r, 1), :] += g_ref[pl.ds(e, 1), :]
                dacc[pl.ds(r, 1), :] += 1.0
            return carry

        lax.fori_loop(0, EB // U, upd, 0)

        @pl.when(step == NSTEP - 1)
        def _():
            acc_t = ((accs[0][...] + accs[1][...])
                     + (accs[2][...] + accs[3][...]))
            deg_t = dacc[:, 0:1]
            agr = acc_t / (deg_t + 1e-8)
            h = jnp.tanh(
                jnp.dot(x_ref[...], w1a_ref[...],
                        preferred_element_type=jnp.float32)
                + jnp.dot(agr, w1b_ref[...],
                          preferred_element_type=jnp.float32)
                + b1_ref[...])
            o_ref[...] = (
                jnp.dot(h, w2_ref[...], preferred_element_type=jnp.float32)
                + b2_ref[...])

    return pl.pallas_call(
        body,
        grid=(NSTEP,),
        in_specs=[
            pl.BlockSpec((1, 1, EB), lambda i: (i, 0, 0),
                         memory_space=pltpu.SMEM),
            pl.BlockSpec((EB, D), lambda i: (i, 0)),
            pl.BlockSpec((NP, D), lambda i: (0, 0)),
            pl.BlockSpec((D, D), lambda i: (0, 0)),
            pl.BlockSpec((D, D), lambda i: (0, 0)),
            pl.BlockSpec((1, D), lambda i: (0, 0)),
            pl.BlockSpec((D, D), lambda i: (0, 0)),
            pl.BlockSpec((1, D), lambda i: (0, 0)),
        ],
        out_specs=pl.BlockSpec((NP, D), lambda i: (0, 0)),
        out_shape=jax.ShapeDtypeStruct((NP, D), jnp.float32),
        scratch_shapes=(
            [pltpu.VMEM((NP, D), jnp.float32)] * 4
            + [pltpu.VMEM((NP, DW), jnp.float32)]
        ),
    )(row2, g2, x_news_pad, w1a, w1b, b1, w2, b2)


def kernel(x_news, x_attr, edge_index, W1, b1, W2, b2):
    row = edge_index[0].astype(jnp.int32)
    col = edge_index[1].astype(jnp.int32)
    g = _gather_sc(x_attr[:, None, :], col)
    g2 = g.reshape(E, D)
    x_news_pad = jnp.pad(x_news, ((0, NP - N), (0, 0)))
    out_pad = _segsum_tc(g2, row.reshape(NSTEPS_ROW, 1, -1), x_news_pad,
                         W1[:D], W1[D:], b1[None, :], W2, b2[None, :])
    return out_pad[:N]
